# baseline (speedup 1.0000x reference)
import functools

import numpy as np
import jax
import jax.numpy as jnp
from jax import lax
from jax.experimental import pallas as pl
from jax.experimental.pallas import tpu as pltpu
from jax.experimental.pallas import tpu_sc as plsc

_B = 4096
_ND = 13
_NS = 26
_VOCAB = 100000
_D = 32
_NFM = _NS + 1
_NPAIR = (_NFM * (_NFM - 1)) // 2
_H0 = 512
_H1 = 256
_NF = 32


_NW = 32
_SAMPLES_PER_W = _B // _NW
_RPW = _SAMPLES_PER_W * _NF
_SUP = 1024
_NSUP = _RPW // _SUP
_LCH = 32
_NLCH = _SUP // _LCH


def _sc_gather_body(table_hbm, idx_hbm, out_hbm, idx_v, g_a, g_b, o_a, o_b,
                    sem_a, sem_b):
    nc = 2
    wid = lax.axis_index("s") * nc + lax.axis_index("c")
    iota = jnp.arange(16, dtype=jnp.int32)

    def fire(base, g_v, sem):
        descs, subs = [], []
        for l in range(_LCH):
            if l % 16 == 0:
                rv = idx_v[pl.ds(base + l, 16)]
            r = rv[l % 16]
            g8 = pl.multiple_of((r >> 3) << 3, 8)
            subs.append(r & 7)
            descs.append(
                pltpu.async_copy(
                    table_hbm.at[pl.ds(g8, 8)],
                    g_v.at[pl.ds(l * 8, 8)],
                    sem,
                )
            )
        return descs, subs

    def finish(descs, subs, g_v, o_v, out_row):
        for d in descs:
            d.wait()
        for l in range(_LCH):
            row = jnp.full((16,), l * 8, jnp.int32) + subs[l]
            dst = jnp.full((16,), l, jnp.int32)
            for h in range(2):
                w = iota + h * 16
                vals = plsc.load_gather(g_v, [row, w])
                plsc.store_scatter(o_v, [dst, w], vals)
        pltpu.sync_copy(o_v, out_hbm.at[pl.ds(out_row, _LCH)])

    @pl.loop(0, _NSUP)
    def _sup(c):
        pltpu.sync_copy(
            idx_hbm.at[pl.ds(wid * _RPW + c * _SUP, _SUP)], idx_v)
        row0 = wid * _RPW + c * _SUP

        @pl.loop(0, _NLCH // 2)
        def _pair(p):
            s0 = p * 2
            da, sa = fire(s0 * _LCH, g_a, sem_a)
            db, sb = fire((s0 + 1) * _LCH, g_b, sem_b)
            finish(da, sa, g_a, o_a, row0 + s0 * _LCH)
            finish(db, sb, g_b, o_b, row0 + (s0 + 1) * _LCH)


def _sc_gather(emb_tables, idx_pad):
    mesh = plsc.VectorSubcoreMesh(core_axis_name="c", subcore_axis_name="s")
    k = pl.kernel(
        _sc_gather_body,
        out_type=jax.ShapeDtypeStruct((_B * _NF, _D), jnp.float32),
        mesh=mesh,
        scratch_types=[
            pltpu.VMEM((_SUP,), jnp.int32),
            pltpu.VMEM((_LCH * 8, _D), jnp.float32),
            pltpu.VMEM((_LCH * 8, _D), jnp.float32),
            pltpu.VMEM((_LCH, _D), jnp.float32),
            pltpu.VMEM((_LCH, _D), jnp.float32),
            pltpu.SemaphoreType.DMA,
            pltpu.SemaphoreType.DMA,
        ],
        compiler_params=pltpu.CompilerParams(needs_layout_passes=False),
    )
    return k(emb_tables, idx_pad)


_BLK = 512


def _tc_body(dense_ref, fm_ref,
             wd0_ref, bd0_ref, wd1_ref, bd1_ref, wd2_ref, bd2_ref,
             wff_ref, wfd_ref, wdd_ref, bc0_ref,
             wc1_ref, bc1_ref, wc2_ref, bc2_ref,
             out_ref):
    f32 = jnp.float32
    x = dense_ref[...]
    h = jnp.maximum(jnp.dot(x, wd0_ref[...], preferred_element_type=f32)
                    + bd0_ref[...], 0.0)
    h = jnp.maximum(jnp.dot(h, wd1_ref[...], preferred_element_type=f32)
                    + bd1_ref[...], 0.0)
    dense = jnp.maximum(jnp.dot(h, wd2_ref[...], preferred_element_type=f32)
                        + bd2_ref[...], 0.0)

    fm = fm_ref[...]
    z = lax.dot_general(fm, fm, (((2,), (2,)), ((0,), (0,))),
                        preferred_element_type=f32)
    zflat = z.reshape(_BLK, _NF * _NF)
    zfd = lax.dot_general(fm, dense, (((2,), (1,)), ((0,), (0,))),
                          preferred_element_type=f32)

    acc = jnp.dot(zflat, wff_ref[...], preferred_element_type=f32)
    acc = acc + jnp.dot(zfd, wfd_ref[...], preferred_element_type=f32)
    acc = acc + jnp.dot(dense, wdd_ref[...], preferred_element_type=f32)
    c = jnp.maximum(acc + bc0_ref[...], 0.0)
    c = jnp.maximum(jnp.dot(c, wc1_ref[...], preferred_element_type=f32)
                    + bc1_ref[...], 0.0)
    logit = jnp.dot(c, wc2_ref[...], preferred_element_type=f32) + bc2_ref[...]
    out_ref[...] = 1.0 / (1.0 + jnp.exp(-logit))


def _tc_forward(dense_x, fm5, wd0, bd0, wd1, bd1, wd2, bd2,
                wff, wfd, wdd, bc0, wc1, bc1, wc2, bc2):
    nblk = _B // _BLK
    full = lambda *shape: pl.BlockSpec(shape, lambda i: (0,) * len(shape))
    return pl.pallas_call(
        _tc_body,
        grid=(nblk,),
        in_specs=[
            pl.BlockSpec((_BLK, _ND), lambda i: (i, 0)),
            pl.BlockSpec((_BLK, _NF, _D), lambda i: (i, 0, 0)),
            full(_ND, _H0), full(1, _H0),
            full(_H0, _H1), full(1, _H1),
            full(_H1, _D), full(1, _D),
            full(_NF * _NF, _H0), full(_NF, _H0), full(_D, _H0), full(1, _H0),
            full(_H0, _H1), full(1, _H1),
            full(_H1, 1), full(1, 1),
        ],
        out_specs=pl.BlockSpec((_BLK, 1), lambda i: (i, 0)),
        out_shape=jax.ShapeDtypeStruct((_B, 1), jnp.float32),
        compiler_params=pltpu.CompilerParams(
            dimension_semantics=("arbitrary",),
        ),
    )(dense_x, fm5, wd0, bd0, wd1, bd1, wd2, bd2,
      wff, wfd, wdd, bc0, wc1, bc1, wc2, bc2)


_IU0, _IU1 = np.triu_indices(_NFM, 1)
_FF_MASK = (_IU0 < _NS) & (_IU1 < _NS)
_FF_DEST = (_IU0[_FF_MASK] * _NF + _IU1[_FF_MASK]).astype(np.int32)
_FF_SRC = np.nonzero(_FF_MASK)[0].astype(np.int32)
_FD_SRC = np.nonzero(~_FF_MASK)[0].astype(np.int32)


def kernel(dense_x, sparse_idx, emb_tables,
           W_d0, b_d0, W_d1, b_d1, W_d2, b_d2,
           W_c0, b_c0, W_c1, b_c1, W_c2, b_c2):
    offsets = jnp.arange(_NS, dtype=sparse_idx.dtype) * _VOCAB
    flat2d = sparse_idx + offsets[None, :]
    filler = (jnp.arange(_B * (_NF - _NS), dtype=jnp.int32)
              .reshape(_B, _NF - _NS) * 997) % (_NS * _VOCAB)
    idx_pad = jnp.concatenate([flat2d, filler], axis=1).reshape(-1)
    fm_flat = _sc_gather(emb_tables, idx_pad)
    fm5 = fm_flat.reshape(_B, _NF, _D)

    wff = jnp.zeros((_NF * _NF, _H0), jnp.float32).at[_FF_DEST].set(
        W_c0[_FF_SRC])
    wfd = jnp.zeros((_NF, _H0), jnp.float32).at[:_NS].set(W_c0[_FD_SRC])
    wdd = W_c0[_NPAIR:]

    return _tc_forward(
        dense_x, fm5,
        W_d0, b_d0.reshape(1, -1), W_d1, b_d1.reshape(1, -1),
        W_d2, b_d2.reshape(1, -1),
        wff, wfd, wdd, b_c0.reshape(1, -1),
        W_c1, b_c1.reshape(1, -1), W_c2, b_c2.reshape(1, -1))


# speedup vs baseline: 1.0475x; 1.0475x over previous
import functools

import numpy as np
import jax
import jax.numpy as jnp
from jax import lax
from jax.experimental import pallas as pl
from jax.experimental.pallas import tpu as pltpu
from jax.experimental.pallas import tpu_sc as plsc

_B = 4096
_ND = 13
_NS = 26
_VOCAB = 100000
_D = 32
_NFM = _NS + 1
_NPAIR = (_NFM * (_NFM - 1)) // 2
_H0 = 512
_H1 = 256
_NF = 32


_NW = 32
_SAMPLES_PER_W = _B // _NW
_RPW = _SAMPLES_PER_W * _NF
_LPW = _SAMPLES_PER_W * _NS
_SUP = 832
_NSUP = _LPW // _SUP
_SCH = 4
_LCH = _SCH * _NS
_OCH = _SCH * _NF
_NLCH = _SUP // _LCH


def _sc_gather_body(table_hbm, idx_hbm, out_hbm, idx_v, g_v, o_v, sem):
    nc = 2
    wid = lax.axis_index("s") * nc + lax.axis_index("c")
    iota = jnp.arange(16, dtype=jnp.int32)
    zeros16 = jnp.zeros((16,), jnp.float32)

    for sl in range(_SCH):
        for i in range(_NS, _NF):
            dst = jnp.full((16,), sl * _NF + i, jnp.int32)
            for h in range(2):
                plsc.store_scatter(o_v, [dst, iota + h * 16], zeros16)

    def fire(base, g_v, sem):
        descs = []
        subs = []
        rv = None
        for l in range(_LCH):
            if l % 16 == 0:
                rv = idx_v[pl.ds(base + l, 16)]
            r = rv[l % 16]
            g8 = pl.multiple_of((r >> 3) << 3, 8)
            subs.append(r & 7)
            descs.append(
                pltpu.async_copy(
                    table_hbm.at[pl.ds(g8, 8)],
                    g_v.at[pl.ds(l * 8, 8)],
                    sem,
                )
            )
        return descs, subs

    def finish(descs, subs, g_v, o_v, out_row):
        for d in descs:
            d.wait()
        for l in range(_LCH):
            row = jnp.full((16,), l * 8, jnp.int32) + subs[l]
            dst = jnp.full((16,), (l // _NS) * _NF + l % _NS, jnp.int32)
            for h in range(2):
                w = iota + h * 16
                vals = plsc.load_gather(g_v, [row, w])
                plsc.store_scatter(o_v, [dst, w], vals)
        pltpu.sync_copy(o_v, out_hbm.at[pl.ds(out_row, _OCH)])

    @pl.loop(0, _NSUP)
    def _sup(c):
        pltpu.sync_copy(
            idx_hbm.at[pl.ds(wid * _LPW + c * _SUP, _SUP)], idx_v)
        row0 = wid * (_SAMPLES_PER_W * _NF) + c * (_SUP // _NS) * _NF

        @pl.loop(0, _NLCH)
        def _lchunk(s):
            da, sa = fire(s * _LCH, g_v, sem)
            finish(da, sa, g_v, o_v, row0 + s * _OCH)


def _sc_gather(emb_tables, idx_flat):
    mesh = plsc.VectorSubcoreMesh(core_axis_name="c", subcore_axis_name="s")
    k = pl.kernel(
        _sc_gather_body,
        out_type=jax.ShapeDtypeStruct((_B * _NF, _D), jnp.float32),
        mesh=mesh,
        scratch_types=[
            pltpu.VMEM((_SUP,), jnp.int32),
            pltpu.VMEM((_LCH * 8, _D), jnp.float32),
            pltpu.VMEM((_OCH, _D), jnp.float32),
            pltpu.SemaphoreType.DMA,
        ],
        compiler_params=pltpu.CompilerParams(needs_layout_passes=False),
    )
    return k(emb_tables, idx_flat)


_BLK = 512


def _tc_body(dense_ref, fm_ref,
             wd0_ref, bd0_ref, wd1_ref, bd1_ref, wd2_ref, bd2_ref,
             wff_ref, wfd_ref, wdd_ref, bc0_ref,
             wc1_ref, bc1_ref, wc2_ref, bc2_ref,
             out_ref):
    f32 = jnp.float32
    x = dense_ref[...]
    h = jnp.maximum(jnp.dot(x, wd0_ref[...], preferred_element_type=f32)
                    + bd0_ref[...], 0.0)
    h = jnp.maximum(jnp.dot(h, wd1_ref[...], preferred_element_type=f32)
                    + bd1_ref[...], 0.0)
    dense = jnp.maximum(jnp.dot(h, wd2_ref[...], preferred_element_type=f32)
                        + bd2_ref[...], 0.0)

    fm = fm_ref[...]
    z = lax.dot_general(fm, fm, (((2,), (2,)), ((0,), (0,))),
                        preferred_element_type=f32)
    zflat = z.reshape(_BLK, _NF * _NF)
    zfd = lax.dot_general(fm, dense, (((2,), (1,)), ((0,), (0,))),
                          preferred_element_type=f32)

    acc = jnp.dot(zflat, wff_ref[...], preferred_element_type=f32)
    acc = acc + jnp.dot(zfd, wfd_ref[...], preferred_element_type=f32)
    acc = acc + jnp.dot(dense, wdd_ref[...], preferred_element_type=f32)
    c = jnp.maximum(acc + bc0_ref[...], 0.0)
    c = jnp.maximum(jnp.dot(c, wc1_ref[...], preferred_element_type=f32)
                    + bc1_ref[...], 0.0)
    logit = jnp.dot(c, wc2_ref[...], preferred_element_type=f32) + bc2_ref[...]
    out_ref[...] = 1.0 / (1.0 + jnp.exp(-logit))


def _tc_forward(dense_x, fm5, wd0, bd0, wd1, bd1, wd2, bd2,
                wff, wfd, wdd, bc0, wc1, bc1, wc2, bc2):
    nblk = _B // _BLK
    full = lambda *shape: pl.BlockSpec(shape, lambda i: (0,) * len(shape))
    return pl.pallas_call(
        _tc_body,
        grid=(nblk,),
        in_specs=[
            pl.BlockSpec((_BLK, _ND), lambda i: (i, 0)),
            pl.BlockSpec((_BLK, _NF, _D), lambda i: (i, 0, 0)),
            full(_ND, _H0), full(1, _H0),
            full(_H0, _H1), full(1, _H1),
            full(_H1, _D), full(1, _D),
            full(_NF * _NF, _H0), full(_NF, _H0), full(_D, _H0), full(1, _H0),
            full(_H0, _H1), full(1, _H1),
            full(_H1, 1), full(1, 1),
        ],
        out_specs=pl.BlockSpec((_BLK, 1), lambda i: (i, 0)),
        out_shape=jax.ShapeDtypeStruct((_B, 1), jnp.float32),
        compiler_params=pltpu.CompilerParams(
            dimension_semantics=("arbitrary",),
        ),
    )(dense_x, fm5, wd0, bd0, wd1, bd1, wd2, bd2,
      wff, wfd, wdd, bc0, wc1, bc1, wc2, bc2)


_IU0, _IU1 = np.triu_indices(_NFM, 1)
_FF_MASK = (_IU0 < _NS) & (_IU1 < _NS)
_FF_DEST = (_IU0[_FF_MASK] * _NF + _IU1[_FF_MASK]).astype(np.int32)
_FF_SRC = np.nonzero(_FF_MASK)[0].astype(np.int32)
_FD_SRC = np.nonzero(~_FF_MASK)[0].astype(np.int32)


def kernel(dense_x, sparse_idx, emb_tables,
           W_d0, b_d0, W_d1, b_d1, W_d2, b_d2,
           W_c0, b_c0, W_c1, b_c1, W_c2, b_c2):
    offsets = jnp.arange(_NS, dtype=sparse_idx.dtype) * _VOCAB
    idx_flat = (sparse_idx + offsets[None, :]).reshape(-1)
    fm_flat = _sc_gather(emb_tables, idx_flat)
    fm5 = fm_flat.reshape(_B, _NF, _D)

    wff = jnp.zeros((_NF * _NF, _H0), jnp.float32).at[_FF_DEST].set(
        W_c0[_FF_SRC])
    wfd = jnp.zeros((_NF, _H0), jnp.float32).at[:_NS].set(W_c0[_FD_SRC])
    wdd = W_c0[_NPAIR:]

    return _tc_forward(
        dense_x, fm5,
        W_d0, b_d0.reshape(1, -1), W_d1, b_d1.reshape(1, -1),
        W_d2, b_d2.reshape(1, -1),
        wff, wfd, wdd, b_c0.reshape(1, -1),
        W_c1, b_c1.reshape(1, -1), W_c2, b_c2.reshape(1, -1))
